# Initial kernel scaffold; baseline (speedup 1.0000x reference)
import jax
import jax.numpy as jnp
from jax.experimental import pallas as pl


def kernel(x, conv1_w, conv1_b, conv2_w, conv2_b, conv3_w, conv3_b, fc1_w, fc1_b, fc2_w, fc2_b, fc3_w, fc3_b, fc4_w, fc4_b, deconv1_w, deconv1_b, deconv2_w, deconv2_b, deconv3_w, deconv3_b):
    raise NotImplementedError("write your pallas kernel here")


# speedup vs baseline: 1.3828x; 1.3828x over previous
import functools

import jax
import jax.numpy as jnp
from jax.experimental import pallas as pl
from jax.experimental.pallas import tpu as pltpu

_SLOPE = 0.01


def _ru(x, m):
    return -(-x // m) * m


def _mm_bias_act_kernel(x_ref, w_ref, b_ref, o_ref, *, act):
    acc = jnp.dot(x_ref[...], w_ref[...], preferred_element_type=jnp.float32)
    acc = acc + b_ref[...]
    if act:
        acc = jnp.where(acc > 0.0, acc, _SLOPE * acc)
    o_ref[...] = acc.astype(o_ref.dtype)


def _mm_plain_kernel(x_ref, w_ref, o_ref):
    acc = jnp.dot(x_ref[...], w_ref[...], preferred_element_type=jnp.float32)
    o_ref[...] = acc.astype(o_ref.dtype)


def _matmul(x, w, b, act, tm=512, tn=512):
    M, K = x.shape
    N = w.shape[1]
    Kp = _ru(K, 128)
    tm = min(tm, _ru(M, 128))
    tn = min(tn, _ru(N, 128))
    Mp = _ru(M, tm)
    Np = _ru(N, tn)
    xp = jnp.pad(x, ((0, Mp - M), (0, Kp - K)))
    wp = jnp.pad(w, ((0, Kp - K), (0, Np - N)))
    grid = (Mp // tm, Np // tn)
    params = pltpu.CompilerParams(dimension_semantics=("parallel", "parallel"))
    if b is not None:
        bp = jnp.pad(b.astype(jnp.float32), (0, Np - N)).reshape(1, Np)
        out = pl.pallas_call(
            functools.partial(_mm_bias_act_kernel, act=act),
            grid=grid,
            in_specs=[
                pl.BlockSpec((tm, Kp), lambda i, j: (i, 0)),
                pl.BlockSpec((Kp, tn), lambda i, j: (0, j)),
                pl.BlockSpec((1, tn), lambda i, j: (0, j)),
            ],
            out_specs=pl.BlockSpec((tm, tn), lambda i, j: (i, j)),
            out_shape=jax.ShapeDtypeStruct((Mp, Np), jnp.bfloat16),
            compiler_params=params,
        )(xp, wp, bp)
    else:
        out = pl.pallas_call(
            _mm_plain_kernel,
            grid=grid,
            in_specs=[
                pl.BlockSpec((tm, Kp), lambda i, j: (i, 0)),
                pl.BlockSpec((Kp, tn), lambda i, j: (0, j)),
            ],
            out_specs=pl.BlockSpec((tm, tn), lambda i, j: (i, j)),
            out_shape=jax.ShapeDtypeStruct((Mp, Np), jnp.bfloat16),
            compiler_params=params,
        )(xp, wp)
    return out[:M, :N]


def _mlp_kernel(h_ref, w1, b1, w2, b2, w3, b3, w4, b4, o_ref):
    def lyr(v, w_ref, b_ref, act):
        y = jnp.dot(v, w_ref[...], preferred_element_type=jnp.float32)
        y = y + b_ref[...]
        if act:
            y = jnp.where(y > 0.0, y, _SLOPE * y)
        return y.astype(jnp.bfloat16)

    h = lyr(h_ref[...], w1, b1, True)
    h = lyr(h, w2, b2, False)
    h = lyr(h, w3, b3, True)
    o_ref[...] = lyr(h, w4, b4, True)


def _mlp(h, fc1_w, fc1_b, fc2_w, fc2_b, fc3_w, fc3_b, fc4_w, fc4_b):
    B = h.shape[0]
    hp = jnp.pad(h, ((0, 0), (0, 1152 - 1120)))
    args = (hp, fc1_w, fc1_b, fc2_w, fc2_b, fc3_w, fc3_b, fc4_w, fc4_b)
    out = pl.pallas_call(
        _mlp_kernel,
        grid=(1,),
        in_specs=[pl.BlockSpec(a.shape, lambda i: (0, 0)) for a in args],
        out_specs=pl.BlockSpec((B, 1152), lambda i: (0, 0)),
        out_shape=jax.ShapeDtypeStruct((B, 1152), jnp.bfloat16),
        compiler_params=pltpu.CompilerParams(
            dimension_semantics=("arbitrary",)),
    )(*args)
    return out[:, :1120]


def _col2im_kernel(y_ref, b_ref, o_ref, acc_ref, *, Kb, s, cs, Hs, Ws, OH,
                   OWL):
    acc_ref[...] = jnp.zeros_like(acc_ref)
    for p in range(Kb):
        for q in range(Kb):
            acc_ref[p * s:p * s + Hs, q * cs:q * cs + Ws] += (
                y_ref[0, p * Kb + q].astype(jnp.float32))
    r = acc_ref[0:OH, 0:OWL] + b_ref[0]
    r = jnp.where(r > 0.0, r, _SLOPE * r)
    o_ref[0] = r.astype(jnp.bfloat16)


def _col2im(y, B, IH, IW, C, K, s, bias):
    Kb = -(-K // s)
    T = Kb * Kb
    OH = (IH - 1) * s + K
    OW = (IW - 1) * s + K
    Hs = IH * s
    Ws = IW * s * C
    OWL = OW * C
    accH = (IH + Kb - 1) * s
    accW = (IW + Kb - 1) * s * C
    yb = y.reshape(B, IH, IW, Kb, Kb, s, s, C)
    yb = yb.transpose(0, 3, 4, 1, 5, 2, 6, 7).reshape(B, T, Hs, Ws)
    bp = jnp.tile(bias.astype(jnp.float32), OW).reshape(1, 1, OWL)
    return pl.pallas_call(
        functools.partial(_col2im_kernel, Kb=Kb, s=s, cs=s * C, Hs=Hs, Ws=Ws,
                          OH=OH, OWL=OWL),
        grid=(B,),
        in_specs=[
            pl.BlockSpec((1, T, Hs, Ws), lambda g: (g, 0, 0, 0)),
            pl.BlockSpec((1, 1, OWL), lambda g: (0, 0, 0)),
        ],
        out_specs=pl.BlockSpec((1, OH, OWL), lambda g: (g, 0, 0)),
        out_shape=jax.ShapeDtypeStruct((B, OH, OWL), jnp.bfloat16),
        scratch_shapes=[pltpu.VMEM((accH, accW), jnp.float32)],
        compiler_params=pltpu.CompilerParams(
            dimension_semantics=("parallel",)),
    )(yb, bp)


def _dec3_bce_kernel(y_ref, b_ref, t_ref, o_ref, acc_ref):
    acc_ref[...] = jnp.zeros_like(acc_ref)
    for p in range(2):
        for q in range(2):
            acc_ref[p * 5:p * 5 + 235, q * 5:q * 5 + 315] += (
                y_ref[0, p * 2 + q].astype(jnp.float32))
    r = acc_ref[...] + b_ref[0]
    t = t_ref[0]
    bce = jnp.maximum(r, 0.0) - r * t + jnp.log1p(jnp.exp(-jnp.abs(r)))
    o_ref[...] = jnp.sum(bce) * jnp.ones((1, 1, 128), jnp.float32)


def _dec3_bce(y, x, bias, B):
    G = B * 3
    yb = y.reshape(B, 47, 63, 3, 2, 2, 5, 5)
    yb = yb.transpose(0, 3, 4, 5, 1, 6, 2, 7).reshape(G, 4, 235, 315)
    t = x.reshape(G, 240, 320)
    bp = jnp.broadcast_to(
        jnp.tile(bias.astype(jnp.float32), B)[:, None, None], (G, 1, 320))
    parts = pl.pallas_call(
        _dec3_bce_kernel,
        grid=(G,),
        in_specs=[
            pl.BlockSpec((1, 4, 235, 315), lambda g: (g, 0, 0, 0)),
            pl.BlockSpec((1, 1, 320), lambda g: (g, 0, 0)),
            pl.BlockSpec((1, 240, 320), lambda g: (g, 0, 0)),
        ],
        out_specs=pl.BlockSpec((1, 1, 128), lambda g: (g, 0, 0)),
        out_shape=jax.ShapeDtypeStruct((G, 1, 128), jnp.float32),
        scratch_shapes=[pltpu.VMEM((240, 320), jnp.float32)],
        compiler_params=pltpu.CompilerParams(
            dimension_semantics=("parallel",)),
    )(yb, bp, t)
    return jnp.sum(parts[:, 0, 0]) / (G * 240.0 * 320.0)


def _im2col(h, K, s):
    B, H, W, C = h.shape
    OH = (H - K) // s + 1
    OW = (W - K) // s + 1
    cols = [h[:, kh:kh + s * OH:s, kw:kw + s * OW:s, :]
            for kh in range(K) for kw in range(K)]
    p = jnp.stack(cols, axis=-1)
    return p.reshape(B * OH * OW, C * K * K)


def kernel(x, conv1_w, conv1_b, conv2_w, conv2_b, conv3_w, conv3_b,
           fc1_w, fc1_b, fc2_w, fc2_b, fc3_w, fc3_b, fc4_w, fc4_b,
           deconv1_w, deconv1_b, deconv2_w, deconv2_b, deconv3_w, deconv3_b):
    B = x.shape[0]
    xb = x.astype(jnp.bfloat16)

    s2d = xb.reshape(B, 3, 48, 5, 64, 5).transpose(0, 2, 4, 3, 5, 1)
    s2d = s2d.reshape(B, 48, 64, 75)
    patches = jnp.concatenate(
        [s2d[:, dh:dh + 47, dw:dw + 63, :] for dh in (0, 1) for dw in (0, 1)],
        axis=-1)
    h = _matmul(patches.reshape(B * 47 * 63, 300), conv1_w, conv1_b, True)
    h = h.reshape(B, 47, 63, 32)

    h = _matmul(_im2col(h, 7, 4), conv2_w, conv2_b, True)
    h = h.reshape(B, 11, 15, 64)

    h = _matmul(_im2col(h, 3, 2), conv3_w, conv3_b, True)

    d = _mlp(h.reshape(B, 1120), fc1_w, fc1_b, fc2_w, fc2_b,
             fc3_w, fc3_b, fc4_w, fc4_b)

    y = _matmul(d.reshape(B * 35, 32), deconv1_w, None, False)
    d1 = _col2im(y, B, 5, 7, 64, 3, 2, deconv1_b)

    y = _matmul(d1.reshape(B * 165, 64), deconv2_w, None, False)
    d2 = _col2im(y, B, 11, 15, 64, 7, 4, deconv2_b)

    y = _matmul(d2.reshape(B * 47 * 63, 64), deconv3_w, None, False)
    recon_loss = _dec3_bce(y, x, deconv3_b, B)
    return recon_loss, recon_loss
